# Initial kernel scaffold; baseline (speedup 1.0000x reference)
import jax
import jax.numpy as jnp
from jax.experimental import pallas as pl


def kernel(atom, bond, edge_index, Wa1, ba1, ga, bta, Wa2, ba2, Wb1, bb1, gb, btb, Wb2, bb2):
    raise NotImplementedError("write your pallas kernel here")


# speedup vs baseline: 2.7155x; 2.7155x over previous
import jax
import jax.numpy as jnp
from jax import lax
from jax.experimental import pallas as pl
from jax.experimental.pallas import tpu as pltpu
from jax.experimental.pallas import tpu_sc as plsc

F32 = jnp.float32
EPS = 1e-5
NC = 2
NS = 16
NW = NC * NS
LANES = 16
CH = 80
BE = 2560


def _sc_mesh():
    return plsc.VectorSubcoreMesh(
        core_axis_name="c", subcore_axis_name="s",
        num_cores=NC, num_subcores=NS)


def _fill_zero(buf, rows, cols):
    zero = jnp.zeros((LANES,), F32)

    def body(r, carry):
        for j in range(cols // LANES):
            buf[r, pl.ds(j * LANES, LANES)] = zero
        return carry

    lax.fori_loop(0, rows, body, 0)


def _zero_acc_rows(src_buf, acc, base, nrows):
    nfull, rem = divmod(nrows, CH)
    for k in range(nfull):
        pltpu.sync_copy(src_buf, acc.at[pl.ds(base + k * CH, CH)])
    if rem:
        pltpu.sync_copy(src_buf.at[pl.ds(0, rem)],
                        acc.at[pl.ds(base + nfull * CH, rem)])


def _pad_rows(N):
    nt = -(-N // NS)
    nt = -(-nt // 8) * 8
    return nt * NS, nt


def _make_gather_segsum(N, E, D):
    EW = E // NW
    QN = EW // CH
    NP, NT = _pad_rows(N)

    NG = -(-QN // 8)

    def body(atom, src3, dst3, sumh, part, src_v, dst_v, a_s, a_d, acc, sem):
        c = lax.axis_index("c")
        s = lax.axis_index("s")
        wid = c * NS + s
        _fill_zero(a_d, CH, D)
        _zero_acc_rows(a_d, acc, s * NT, NT)
        plsc.subcore_barrier()

        def group(g, carry):
            pltpu.sync_copy(src3.at[wid, pl.ds(g * 8, 8)], src_v)
            pltpu.sync_copy(dst3.at[wid, pl.ds(g * 8, 8)], dst_v)

            def chunk(k, carry2):
                q = g * 8 + k

                @pl.when(q < QN)
                def _():
                    ebase = wid * EW + q * CH
                    d1 = pltpu.async_copy(atom.at[src_v.at[k]], a_s, sem)
                    d2 = pltpu.async_copy(atom.at[dst_v.at[k]], a_d, sem)
                    d1.wait()
                    d2.wait()

                    def row(r, carry3):
                        for j in range(D // LANES):
                            sl = pl.ds(j * LANES, LANES)
                            a_d[r, sl] = a_s[r, sl] + a_d[r, sl]
                        return carry3

                    lax.fori_loop(0, CH, row, 0)
                    pltpu.sync_copy(a_d, sumh.at[pl.ds(ebase, CH)])
                    pltpu.sync_copy(a_s, acc.at[dst_v.at[k]], add=True)

                return carry2

            lax.fori_loop(0, 8, chunk, carry)
            return carry

        lax.fori_loop(0, NG, group, 0)
        plsc.subcore_barrier()
        pltpu.sync_copy(acc.at[pl.ds(s * NT, NT)],
                        part.at[c, pl.ds(s * NT, NT)])

    return pl.kernel(
        body,
        out_type=[jax.ShapeDtypeStruct((E, D), F32),
                  jax.ShapeDtypeStruct((NC, NP, D), F32)],
        mesh=_sc_mesh(),
        scratch_types=[
            pltpu.VMEM((8, CH), jnp.int32),
            pltpu.VMEM((8, CH), jnp.int32),
            pltpu.VMEM((CH, D), F32),
            pltpu.VMEM((CH, D), F32),
            pltpu.VMEM_SHARED((NP, D), F32),
            pltpu.SemaphoreType.DMA,
        ],
    )


def _make_segsum(N, E, D):
    EW = E // NW
    QN = EW // CH
    NP, NT = _pad_rows(N)

    NG = -(-QN // 8)

    def body(evals, dst3, part, dst_v, e_v, acc):
        c = lax.axis_index("c")
        s = lax.axis_index("s")
        wid = c * NS + s
        _fill_zero(e_v, CH, D)
        _zero_acc_rows(e_v, acc, s * NT, NT)
        plsc.subcore_barrier()

        def group(g, carry):
            pltpu.sync_copy(dst3.at[wid, pl.ds(g * 8, 8)], dst_v)

            def chunk(k, carry2):
                q = g * 8 + k

                @pl.when(q < QN)
                def _():
                    ebase = wid * EW + q * CH
                    pltpu.sync_copy(evals.at[pl.ds(ebase, CH)], e_v)
                    pltpu.sync_copy(e_v, acc.at[dst_v.at[k]], add=True)

                return carry2

            lax.fori_loop(0, 8, chunk, carry)
            return carry

        lax.fori_loop(0, NG, group, 0)
        plsc.subcore_barrier()
        pltpu.sync_copy(acc.at[pl.ds(s * NT, NT)],
                        part.at[c, pl.ds(s * NT, NT)])

    return pl.kernel(
        body,
        out_type=jax.ShapeDtypeStruct((NC, NP, D), F32),
        mesh=_sc_mesh(),
        scratch_types=[
            pltpu.VMEM((8, CH), jnp.int32),
            pltpu.VMEM((CH, D), F32),
            pltpu.VMEM_SHARED((NP, D), F32),
        ],
    )


def _make_edge_mlp(E, D, H):
    NB = E // BE

    def body(sumh_ref, bond_ref, w1a, w1b, b1, g, bt, w2, b2, e_ref,
             s1, s2, ca, cb):
        p = pl.program_id(0)
        j = pl.program_id(1)
        y = (jnp.dot(sumh_ref[...], w1a[...], preferred_element_type=F32)
             + jnp.dot(bond_ref[...], w1b[...], preferred_element_type=F32)
             + b1[...])

        @pl.when(p == 0)
        def _():
            @pl.when(j == 0)
            def _():
                s1[...] = jnp.zeros_like(s1)
                s2[...] = jnp.zeros_like(s2)

            s1[...] += jnp.sum(y, axis=0, keepdims=True)
            s2[...] += jnp.sum(y * y, axis=0, keepdims=True)

            @pl.when(j == NB - 1)
            def _():
                mu = s1[...] / E
                var = s2[...] / E - mu * mu
                a = g[...] * lax.rsqrt(var + EPS)
                ca[...] = a
                cb[...] = bt[...] - mu * a

        @pl.when(p == 1)
        def _():
            yh = jnp.maximum(y * ca[...] + cb[...], 0.0)
            e_ref[...] = jnp.dot(yh, w2[...], preferred_element_type=F32) + b2[...]

    return pl.pallas_call(
        body,
        grid=(2, NB),
        in_specs=[
            pl.BlockSpec((BE, D), lambda p, j: (j, 0)),
            pl.BlockSpec((BE, D), lambda p, j: (j, 0)),
            pl.BlockSpec((D, H), lambda p, j: (0, 0)),
            pl.BlockSpec((D, H), lambda p, j: (0, 0)),
            pl.BlockSpec((1, H), lambda p, j: (0, 0)),
            pl.BlockSpec((1, H), lambda p, j: (0, 0)),
            pl.BlockSpec((1, H), lambda p, j: (0, 0)),
            pl.BlockSpec((H, D), lambda p, j: (0, 0)),
            pl.BlockSpec((1, D), lambda p, j: (0, 0)),
        ],
        out_specs=pl.BlockSpec((BE, D), lambda p, j: (jnp.where(p == 0, 0, j), 0)),
        out_shape=jax.ShapeDtypeStruct((E, D), F32),
        scratch_shapes=[pltpu.VMEM((1, H), F32)] * 4,
        compiler_params=pltpu.CompilerParams(
            dimension_semantics=("arbitrary", "arbitrary")),
    )


def _make_node_mlp(N, D, H):
    def body(ph, pe, w1a, w1b, b1, g, bt, w2, b2, h_ref):
        sh = ph[0, :N] + ph[1, :N]
        se = pe[0, :N] + pe[1, :N]
        y = (jnp.dot(sh, w1a[...], preferred_element_type=F32)
             + jnp.dot(se, w1b[...], preferred_element_type=F32)
             + b1[...])
        mu = jnp.mean(y, axis=0, keepdims=True)
        var = jnp.mean(y * y, axis=0, keepdims=True) - mu * mu
        yh = jnp.maximum((y - mu) * (g[...] * lax.rsqrt(var + EPS)) + bt[...], 0.0)
        h_ref[...] = jnp.dot(yh, w2[...], preferred_element_type=F32) + b2[...]

    return pl.pallas_call(
        body,
        out_shape=jax.ShapeDtypeStruct((N, D), F32),
    )


def kernel(atom, bond, edge_index, Wa1, ba1, ga, bta, Wa2, ba2,
           Wb1, bb1, gb, btb, Wb2, bb2):
    N, D = atom.shape
    E = bond.shape[0]
    H = Wb1.shape[1]
    assert E % (NW * CH) == 0 and E % BE == 0
    QN = E // (NW * CH)
    pad = -(-QN // 8) * 8 - QN

    src3 = jnp.pad(edge_index[0].reshape(NW, QN, CH), ((0, 0), (0, pad), (0, 0)))
    dst3 = jnp.pad(edge_index[1].reshape(NW, QN, CH), ((0, 0), (0, pad), (0, 0)))

    sumh, ph = _make_gather_segsum(N, E, D)(atom, src3, dst3)
    e = _make_edge_mlp(E, D, H)(
        sumh, bond, Wb1[:D], Wb1[D:], bb1.reshape(1, H), gb.reshape(1, H),
        btb.reshape(1, H), Wb2, bb2.reshape(1, D))
    pe = _make_segsum(N, E, D)(e, dst3)
    h = _make_node_mlp(N, D, H)(
        ph, pe, Wa1[:D], Wa1[D:], ba1.reshape(1, H), ga.reshape(1, H),
        bta.reshape(1, H), Wa2, ba2.reshape(1, D))
    return h, e


# speedup vs baseline: 2.7163x; 1.0003x over previous
import jax
import jax.numpy as jnp
from jax import lax
from jax.experimental import pallas as pl
from jax.experimental.pallas import tpu as pltpu
from jax.experimental.pallas import tpu_sc as plsc

F32 = jnp.float32
EPS = 1e-5
NC = 2
NS = 16
NW = NC * NS
LANES = 16
CH = 80
BE = 2560


def _sc_mesh():
    return plsc.VectorSubcoreMesh(
        core_axis_name="c", subcore_axis_name="s",
        num_cores=NC, num_subcores=NS)


def _fill_zero(buf, rows, cols):
    zero = jnp.zeros((LANES,), F32)

    def body(r, carry):
        for j in range(cols // LANES):
            buf[r, pl.ds(j * LANES, LANES)] = zero
        return carry

    lax.fori_loop(0, rows, body, 0)


def _zero_acc_rows(src_buf, acc, base, nrows):
    nfull, rem = divmod(nrows, CH)
    for k in range(nfull):
        pltpu.sync_copy(src_buf, acc.at[pl.ds(base + k * CH, CH)])
    if rem:
        pltpu.sync_copy(src_buf.at[pl.ds(0, rem)],
                        acc.at[pl.ds(base + nfull * CH, rem)])


def _pad_rows(N):
    nt = -(-N // NS)
    nt = -(-nt // 8) * 8
    return nt * NS, nt


def _make_gather_segsum(N, E, D):
    EW = E // NW
    QN = EW // CH
    NP, NT = _pad_rows(N)

    NG = -(-QN // 8)

    def body(atom, src3, dst3, sumh, part, src_v, dst_v, a_s, a_d, acc, sem):
        c = lax.axis_index("c")
        s = lax.axis_index("s")
        wid = c * NS + s
        _fill_zero(a_d, CH, D)
        _zero_acc_rows(a_d, acc, s * NT, NT)
        plsc.subcore_barrier()

        def group(g, carry):
            pltpu.sync_copy(src3.at[wid, pl.ds(g * 8, 8)], src_v)
            pltpu.sync_copy(dst3.at[wid, pl.ds(g * 8, 8)], dst_v)

            def chunk(k, carry2):
                q = g * 8 + k

                @pl.when(q < QN)
                def _():
                    ebase = wid * EW + q * CH
                    d1 = pltpu.async_copy(atom.at[src_v.at[k]], a_s, sem)
                    d2 = pltpu.async_copy(atom.at[dst_v.at[k]], a_d, sem)
                    d1.wait()
                    d2.wait()

                    def row(r, carry3):
                        for j in range(D // LANES):
                            sl = pl.ds(j * LANES, LANES)
                            a_d[r, sl] = a_s[r, sl] + a_d[r, sl]
                        return carry3

                    lax.fori_loop(0, CH, row, 0)
                    pltpu.sync_copy(a_d, sumh.at[pl.ds(ebase, CH)])
                    pltpu.sync_copy(a_s, acc.at[dst_v.at[k]], add=True)

                return carry2

            lax.fori_loop(0, 8, chunk, carry)
            return carry

        lax.fori_loop(0, NG, group, 0)
        plsc.subcore_barrier()
        pltpu.sync_copy(acc.at[pl.ds(s * NT, NT)],
                        part.at[c, pl.ds(s * NT, NT)])

    return pl.kernel(
        body,
        out_type=[jax.ShapeDtypeStruct((E, D), F32),
                  jax.ShapeDtypeStruct((NC, NP, D), F32)],
        mesh=_sc_mesh(),
        scratch_types=[
            pltpu.VMEM((8, CH), jnp.int32),
            pltpu.VMEM((8, CH), jnp.int32),
            pltpu.VMEM((CH, D), F32),
            pltpu.VMEM((CH, D), F32),
            pltpu.VMEM_SHARED((NP, D), F32),
            pltpu.SemaphoreType.DMA,
        ],
    )


def _make_segsum(N, E, D):
    EW = E // NW
    QN = EW // CH
    NP, NT = _pad_rows(N)

    NG = -(-QN // 8)

    def body(evals, dst3, part, dst_v, e_v, acc):
        c = lax.axis_index("c")
        s = lax.axis_index("s")
        wid = c * NS + s
        _fill_zero(e_v, CH, D)
        _zero_acc_rows(e_v, acc, s * NT, NT)
        plsc.subcore_barrier()

        def group(g, carry):
            pltpu.sync_copy(dst3.at[wid, pl.ds(g * 8, 8)], dst_v)

            def chunk(k, carry2):
                q = g * 8 + k

                @pl.when(q < QN)
                def _():
                    ebase = wid * EW + q * CH
                    pltpu.sync_copy(evals.at[pl.ds(ebase, CH)], e_v)
                    pltpu.sync_copy(e_v, acc.at[dst_v.at[k]], add=True)

                return carry2

            lax.fori_loop(0, 8, chunk, carry)
            return carry

        lax.fori_loop(0, NG, group, 0)
        plsc.subcore_barrier()
        pltpu.sync_copy(acc.at[pl.ds(s * NT, NT)],
                        part.at[c, pl.ds(s * NT, NT)])

    return pl.kernel(
        body,
        out_type=jax.ShapeDtypeStruct((NC, NP, D), F32),
        mesh=_sc_mesh(),
        scratch_types=[
            pltpu.VMEM((8, CH), jnp.int32),
            pltpu.VMEM((CH, D), F32),
            pltpu.VMEM_SHARED((NP, D), F32),
        ],
    )


def _make_edge_mlp(E, D, H):
    NB = E // BE

    def body(sumh_ref, bond_ref, w1a, w1b, b1, g, bt, w2, b2, e_ref,
             s1, s2, ca, cb):
        p = pl.program_id(0)
        j = pl.program_id(1)
        bf = jnp.bfloat16
        y = (jnp.dot(sumh_ref[...].astype(bf), w1a[...].astype(bf),
                     preferred_element_type=F32)
             + jnp.dot(bond_ref[...].astype(bf), w1b[...].astype(bf),
                       preferred_element_type=F32)
             + b1[...])

        @pl.when(p == 0)
        def _():
            @pl.when(j == 0)
            def _():
                s1[...] = jnp.zeros_like(s1)
                s2[...] = jnp.zeros_like(s2)

            s1[...] += jnp.sum(y, axis=0, keepdims=True)
            s2[...] += jnp.sum(y * y, axis=0, keepdims=True)

            @pl.when(j == NB - 1)
            def _():
                mu = s1[...] / E
                var = s2[...] / E - mu * mu
                a = g[...] * lax.rsqrt(var + EPS)
                ca[...] = a
                cb[...] = bt[...] - mu * a

        @pl.when(p == 1)
        def _():
            yh = jnp.maximum(y * ca[...] + cb[...], 0.0)
            e_ref[...] = jnp.dot(yh.astype(bf), w2[...].astype(bf),
                                 preferred_element_type=F32) + b2[...]

    return pl.pallas_call(
        body,
        grid=(2, NB),
        in_specs=[
            pl.BlockSpec((BE, D), lambda p, j: (j, 0)),
            pl.BlockSpec((BE, D), lambda p, j: (j, 0)),
            pl.BlockSpec((D, H), lambda p, j: (0, 0)),
            pl.BlockSpec((D, H), lambda p, j: (0, 0)),
            pl.BlockSpec((1, H), lambda p, j: (0, 0)),
            pl.BlockSpec((1, H), lambda p, j: (0, 0)),
            pl.BlockSpec((1, H), lambda p, j: (0, 0)),
            pl.BlockSpec((H, D), lambda p, j: (0, 0)),
            pl.BlockSpec((1, D), lambda p, j: (0, 0)),
        ],
        out_specs=pl.BlockSpec((BE, D), lambda p, j: (jnp.where(p == 0, 0, j), 0)),
        out_shape=jax.ShapeDtypeStruct((E, D), F32),
        scratch_shapes=[pltpu.VMEM((1, H), F32)] * 4,
        compiler_params=pltpu.CompilerParams(
            dimension_semantics=("arbitrary", "arbitrary")),
    )


def _make_node_mlp(N, D, H):
    def body(ph, pe, w1a, w1b, b1, g, bt, w2, b2, h_ref):
        bf = jnp.bfloat16
        sh = ph[0, :N] + ph[1, :N]
        se = pe[0, :N] + pe[1, :N]
        y = (jnp.dot(sh.astype(bf), w1a[...].astype(bf),
                     preferred_element_type=F32)
             + jnp.dot(se.astype(bf), w1b[...].astype(bf),
                       preferred_element_type=F32)
             + b1[...])
        mu = jnp.mean(y, axis=0, keepdims=True)
        var = jnp.mean(y * y, axis=0, keepdims=True) - mu * mu
        yh = jnp.maximum((y - mu) * (g[...] * lax.rsqrt(var + EPS)) + bt[...], 0.0)
        h_ref[...] = jnp.dot(yh.astype(bf), w2[...].astype(bf),
                             preferred_element_type=F32) + b2[...]

    return pl.pallas_call(
        body,
        out_shape=jax.ShapeDtypeStruct((N, D), F32),
    )


def kernel(atom, bond, edge_index, Wa1, ba1, ga, bta, Wa2, ba2,
           Wb1, bb1, gb, btb, Wb2, bb2):
    N, D = atom.shape
    E = bond.shape[0]
    H = Wb1.shape[1]
    assert E % (NW * CH) == 0 and E % BE == 0
    QN = E // (NW * CH)
    pad = -(-QN // 8) * 8 - QN

    src3 = jnp.pad(edge_index[0].reshape(NW, QN, CH), ((0, 0), (0, pad), (0, 0)))
    dst3 = jnp.pad(edge_index[1].reshape(NW, QN, CH), ((0, 0), (0, pad), (0, 0)))

    sumh, ph = _make_gather_segsum(N, E, D)(atom, src3, dst3)
    e = _make_edge_mlp(E, D, H)(
        sumh, bond, Wb1[:D], Wb1[D:], bb1.reshape(1, H), gb.reshape(1, H),
        btb.reshape(1, H), Wb2, bb2.reshape(1, D))
    pe = _make_segsum(N, E, D)(e, dst3)
    h = _make_node_mlp(N, D, H)(
        ph, pe, Wa1[:D], Wa1[D:], ba1.reshape(1, H), ga.reshape(1, H),
        bta.reshape(1, H), Wa2, ba2.reshape(1, D))
    return h, e


# speedup vs baseline: 2.9389x; 1.0819x over previous
import jax
import jax.numpy as jnp
from jax import lax
from jax.experimental import pallas as pl
from jax.experimental.pallas import tpu as pltpu
from jax.experimental.pallas import tpu_sc as plsc

F32 = jnp.float32
EPS = 1e-5
NC = 2
NS = 16
NW = NC * NS
LANES = 16
CH = 80
BE = 2560


def _sc_mesh():
    return plsc.VectorSubcoreMesh(
        core_axis_name="c", subcore_axis_name="s",
        num_cores=NC, num_subcores=NS)


def _fill_zero(buf, rows, cols):
    zero = jnp.zeros((LANES,), F32)

    def body(r, carry):
        for j in range(cols // LANES):
            buf[r, pl.ds(j * LANES, LANES)] = zero
        return carry

    lax.fori_loop(0, rows, body, 0)


def _zero_acc_rows(src_buf, acc, base, nrows):
    nfull, rem = divmod(nrows, CH)
    for k in range(nfull):
        pltpu.sync_copy(src_buf, acc.at[pl.ds(base + k * CH, CH)])
    if rem:
        pltpu.sync_copy(src_buf.at[pl.ds(0, rem)],
                        acc.at[pl.ds(base + nfull * CH, rem)])


def _pad_rows(N):
    nt = -(-N // NS)
    nt = -(-nt // 8) * 8
    return nt * NS, nt


def _make_gather_segsum(N, E, D):
    EW = E // NW
    QN = EW // CH
    NP, NT = _pad_rows(N)

    NG = -(-QN // 8)

    def body(atom, src3, dst3, sumh, part, src_v, dst_v, a_s, a_d, acc, sem):
        c = lax.axis_index("c")
        s = lax.axis_index("s")
        wid = c * NS + s
        _fill_zero(a_d, CH, D)
        _zero_acc_rows(a_d, acc, s * NT, NT)
        plsc.subcore_barrier()

        def group(g, carry):
            pltpu.sync_copy(src3.at[wid, pl.ds(g * 8, 8)], src_v)
            pltpu.sync_copy(dst3.at[wid, pl.ds(g * 8, 8)], dst_v)

            def chunk(k, carry2):
                q = g * 8 + k

                @pl.when(q < QN)
                def _():
                    ebase = wid * EW + q * CH
                    d1 = pltpu.async_copy(atom.at[src_v.at[k]], a_s, sem)
                    d2 = pltpu.async_copy(atom.at[dst_v.at[k]], a_d, sem)
                    d1.wait()
                    d2.wait()

                    def row(r, carry3):
                        for j in range(D // LANES):
                            sl = pl.ds(j * LANES, LANES)
                            a_d[r, sl] = a_s[r, sl] + a_d[r, sl]
                        return carry3

                    lax.fori_loop(0, CH, row, 0)
                    pltpu.sync_copy(a_d, sumh.at[pl.ds(ebase, CH)])
                    pltpu.sync_copy(a_s, acc.at[dst_v.at[k]], add=True)

                return carry2

            lax.fori_loop(0, 8, chunk, carry)
            return carry

        lax.fori_loop(0, NG, group, 0)
        plsc.subcore_barrier()
        pltpu.sync_copy(acc.at[pl.ds(s * NT, NT)],
                        part.at[c, pl.ds(s * NT, NT)])

    return pl.kernel(
        body,
        out_type=[jax.ShapeDtypeStruct((E, D), F32),
                  jax.ShapeDtypeStruct((NC, NP, D), F32)],
        mesh=_sc_mesh(),
        scratch_types=[
            pltpu.VMEM((8, CH), jnp.int32),
            pltpu.VMEM((8, CH), jnp.int32),
            pltpu.VMEM((CH, D), F32),
            pltpu.VMEM((CH, D), F32),
            pltpu.VMEM_SHARED((NP, D), F32),
            pltpu.SemaphoreType.DMA,
        ],
    )


def _make_segsum(N, E, D):
    EW = E // NW
    QN = EW // CH
    NP, NT = _pad_rows(N)

    T = -(-QN // 2)

    def body(evals, dst3, part, dst_v, e0, e1, acc, sr0, sr1, sw0, sw1):
        c = lax.axis_index("c")
        s = lax.axis_index("s")
        wid = c * NS + s
        _fill_zero(e0, CH, D)
        _zero_acc_rows(e0, acc, s * NT, NT)
        plsc.subcore_barrier()

        def rd(q, buf, sem):
            pltpu.async_copy(evals.at[pl.ds(wid * EW + q * CH, CH)], buf, sem)

        def dma_wait(buf, sem):
            pltpu.make_async_copy(evals.at[pl.ds(0, CH)], buf, sem).wait()

        def sc(k, buf, sem):
            pltpu.async_copy(buf, acc.at[dst_v.at[k]], sem, add=True)

        rd(0, e0, sr0)

        def pair(t, carry):
            q0 = 2 * t
            q1 = q0 + 1

            @pl.when(t > 0)
            def _():
                dma_wait(e1, sw1)

            @pl.when(lax.rem(t, 4) == 0)
            def _():
                pltpu.sync_copy(dst3.at[wid, pl.ds((t // 4) * 8, 8)], dst_v)

            @pl.when(q1 < QN)
            def _():
                rd(q1, e1, sr1)

            dma_wait(e0, sr0)
            sc(lax.rem(q0, 8), e0, sw0)

            @pl.when(q1 < QN)
            def _():
                dma_wait(e1, sr1)

            dma_wait(e0, sw0)

            @pl.when(q0 + 2 < QN)
            def _():
                rd(q0 + 2, e0, sr0)

            @pl.when(q1 < QN)
            def _():
                sc(lax.rem(q1, 8), e1, sw1)

            return carry

        lax.fori_loop(0, T, pair, 0)
        if QN % 2 == 0:
            dma_wait(e1, sw1)
        plsc.subcore_barrier()
        pltpu.sync_copy(acc.at[pl.ds(s * NT, NT)],
                        part.at[c, pl.ds(s * NT, NT)])

    return pl.kernel(
        body,
        out_type=jax.ShapeDtypeStruct((NC, NP, D), F32),
        mesh=_sc_mesh(),
        scratch_types=[
            pltpu.VMEM((8, CH), jnp.int32),
            pltpu.VMEM((CH, D), F32),
            pltpu.VMEM((CH, D), F32),
            pltpu.VMEM_SHARED((NP, D), F32),
            pltpu.SemaphoreType.DMA,
            pltpu.SemaphoreType.DMA,
            pltpu.SemaphoreType.DMA,
            pltpu.SemaphoreType.DMA,
        ],
    )


def _make_edge_mlp(E, D, H):
    NB = E // BE

    def body(sumh_ref, bond_ref, w1a, w1b, b1, g, bt, w2, b2, e_ref,
             s1, s2, ca, cb):
        p = pl.program_id(0)
        j = pl.program_id(1)
        bf = jnp.bfloat16
        y = (jnp.dot(sumh_ref[...].astype(bf), w1a[...].astype(bf),
                     preferred_element_type=F32)
             + jnp.dot(bond_ref[...].astype(bf), w1b[...].astype(bf),
                       preferred_element_type=F32)
             + b1[...])

        @pl.when(p == 0)
        def _():
            @pl.when(j == 0)
            def _():
                s1[...] = jnp.zeros_like(s1)
                s2[...] = jnp.zeros_like(s2)

            s1[...] += jnp.sum(y, axis=0, keepdims=True)
            s2[...] += jnp.sum(y * y, axis=0, keepdims=True)

            @pl.when(j == NB - 1)
            def _():
                mu = s1[...] / E
                var = s2[...] / E - mu * mu
                a = g[...] * lax.rsqrt(var + EPS)
                ca[...] = a
                cb[...] = bt[...] - mu * a

        @pl.when(p == 1)
        def _():
            yh = jnp.maximum(y * ca[...] + cb[...], 0.0)
            e_ref[...] = jnp.dot(yh.astype(bf), w2[...].astype(bf),
                                 preferred_element_type=F32) + b2[...]

    return pl.pallas_call(
        body,
        grid=(2, NB),
        in_specs=[
            pl.BlockSpec((BE, D), lambda p, j: (j, 0)),
            pl.BlockSpec((BE, D), lambda p, j: (j, 0)),
            pl.BlockSpec((D, H), lambda p, j: (0, 0)),
            pl.BlockSpec((D, H), lambda p, j: (0, 0)),
            pl.BlockSpec((1, H), lambda p, j: (0, 0)),
            pl.BlockSpec((1, H), lambda p, j: (0, 0)),
            pl.BlockSpec((1, H), lambda p, j: (0, 0)),
            pl.BlockSpec((H, D), lambda p, j: (0, 0)),
            pl.BlockSpec((1, D), lambda p, j: (0, 0)),
        ],
        out_specs=pl.BlockSpec((BE, D), lambda p, j: (jnp.where(p == 0, 0, j), 0)),
        out_shape=jax.ShapeDtypeStruct((E, D), F32),
        scratch_shapes=[pltpu.VMEM((1, H), F32)] * 4,
        compiler_params=pltpu.CompilerParams(
            dimension_semantics=("arbitrary", "arbitrary")),
    )


def _make_node_mlp(N, D, H):
    def body(ph, pe, w1a, w1b, b1, g, bt, w2, b2, h_ref):
        bf = jnp.bfloat16
        sh = ph[0, :N] + ph[1, :N]
        se = pe[0, :N] + pe[1, :N]
        y = (jnp.dot(sh.astype(bf), w1a[...].astype(bf),
                     preferred_element_type=F32)
             + jnp.dot(se.astype(bf), w1b[...].astype(bf),
                       preferred_element_type=F32)
             + b1[...])
        mu = jnp.mean(y, axis=0, keepdims=True)
        var = jnp.mean(y * y, axis=0, keepdims=True) - mu * mu
        yh = jnp.maximum((y - mu) * (g[...] * lax.rsqrt(var + EPS)) + bt[...], 0.0)
        h_ref[...] = jnp.dot(yh.astype(bf), w2[...].astype(bf),
                             preferred_element_type=F32) + b2[...]

    return pl.pallas_call(
        body,
        out_shape=jax.ShapeDtypeStruct((N, D), F32),
    )


def kernel(atom, bond, edge_index, Wa1, ba1, ga, bta, Wa2, ba2,
           Wb1, bb1, gb, btb, Wb2, bb2):
    N, D = atom.shape
    E = bond.shape[0]
    H = Wb1.shape[1]
    assert E % (NW * CH) == 0 and E % BE == 0
    QN = E // (NW * CH)
    pad = -(-QN // 8) * 8 - QN

    src3 = jnp.pad(edge_index[0].reshape(NW, QN, CH), ((0, 0), (0, pad), (0, 0)))
    dst3 = jnp.pad(edge_index[1].reshape(NW, QN, CH), ((0, 0), (0, pad), (0, 0)))

    sumh, ph = _make_gather_segsum(N, E, D)(atom, src3, dst3)
    e = _make_edge_mlp(E, D, H)(
        sumh, bond, Wb1[:D], Wb1[D:], bb1.reshape(1, H), gb.reshape(1, H),
        btb.reshape(1, H), Wb2, bb2.reshape(1, D))
    pe = _make_segsum(N, E, D)(e, dst3)
    h = _make_node_mlp(N, D, H)(
        ph, pe, Wa1[:D], Wa1[D:], ba1.reshape(1, H), ga.reshape(1, H),
        bta.reshape(1, H), Wa2, ba2.reshape(1, D))
    return h, e
